# baseline (speedup 1.0000x reference)
import jax
import jax.numpy as jnp
from jax import lax
from jax.experimental import pallas as pl
from jax.experimental.pallas import tpu as pltpu
from jax.experimental.pallas import tpu_sc as plsc

_B, _N, _MC = 8, 1024, 16
_F, _CONV, _NL = 256, 512, 128
_NC, _NS, _L = 2, 16, 16
_NW = _NC * _NS
_RPW = (_B * _N) // _NW
_WPB = _N // _RPW
_CH = 8
_NCHUNK = _RPW // _CH
_NV = _F // _L

_mesh = plsc.VectorSubcoreMesh(core_axis_name="c", subcore_axis_name="s",
                               num_cores=_NC, num_subcores=_NS)


def _sc_body(node_hbm, pnode_hbm, ch_hbm, emb_hbm, p_hbm, s_hbm,
             node_v, pidx_v, ch_v, gidx0, gidx1, rows0, rows1,
             ab0, ab1, st0, st1, sem0, sem1, stsem0, stsem1):
    wid = lax.axis_index("s") * _NC + lax.axis_index("c")
    b = wid // _WPB
    r0 = (wid % _WPB) * _RPW
    base = wid * _RPW

    pltpu.sync_copy(node_hbm.at[b], node_v)
    pltpu.sync_copy(ch_hbm.at[b, pl.ds(r0 * _MC, _RPW * _MC)], ch_v)
    pltpu.sync_copy(pnode_hbm.at[wid], pidx_v)

    def parent_k(k, _):
        pltpu.async_copy(emb_hbm.at[pidx_v.at[k]],
                         rows0.at[pl.ds(0, 128)], sem0).wait()
        pltpu.sync_copy(rows0.at[pl.ds(0, 128)],
                        p_hbm.at[pl.ds(base + k * 128, 128)])
        return 0
    lax.fori_loop(0, 2, parent_k, 0, unroll=True)

    lane0 = lax.iota(jnp.int32, _L) == 0

    def build_idx(c, gidx_v, ab_v):
        def node_body(i, _):
            ch = ch_v[pl.ds((c * _CH + i) * _MC, _L)]
            g = plsc.load_gather(node_v, [ch])
            g = jnp.where(ch == 0, 0, g)
            gidx_v[pl.ds(i * _L, _L)] = g
            ns = plsc.all_reduce_population_count(ch > 0)
            nsf = ns.astype(jnp.float32)
            one = ns == 1
            alpha = jnp.where(one, jnp.float32(0.0), 1.0 / (nsf - 1.0))
            beta = jnp.where(one, jnp.float32(0.5), jnp.float32(0.0))
            iv = jnp.full((_L,), i, jnp.int32)
            plsc.store_scatter(ab_v, [iv], alpha, mask=lane0)
            plsc.store_scatter(ab_v, [jnp.full((_L,), _L + i, jnp.int32)],
                               beta, mask=lane0)
            return 0
        lax.fori_loop(0, _CH, node_body, 0)

    def compute(rows_v, ab_v, st_v):
        ab = ab_v[pl.ds(0, _L)]
        bb = ab_v[pl.ds(_L, _L)]
        for i in range(_CH):
            alphab = jnp.full((_L,), ab[i])
            betab = jnp.full((_L,), bb[i])

            def comp_vreg(v, _):
                t = jnp.zeros((_L,), jnp.float32)
                bacc = jnp.zeros((_L,), jnp.float32)
                r = t
                for j in range(_MC - 1, -1, -1):
                    r = rows_v[i * _MC + j, pl.ds(v * _L, _L)]
                    t = t + r
                    bacc = bacc + t
                s1 = alphab * (bacc - t) + betab * r
                st_v[i, pl.ds(v * _L, _L)] = s1
                st_v[i, pl.ds(_F + v * _L, _L)] = t - s1
                return 0
            lax.fori_loop(0, _NV, comp_vreg, 0, unroll=2)

    build_idx(0, gidx0, ab0)
    pltpu.async_copy(emb_hbm.at[gidx0], rows0, sem0)

    def outer(t, _):
        c0 = 2 * t
        build_idx(c0 + 1, gidx1, ab1)
        pltpu.async_copy(emb_hbm.at[gidx1], rows1, sem1)
        pltpu.make_async_copy(emb_hbm.at[gidx0], rows0, sem0).wait()

        @pl.when(t > 0)
        def _():
            pltpu.make_async_copy(st0, s_hbm.at[pl.ds(0, _CH)], stsem0).wait()
        compute(rows0, ab0, st0)
        pltpu.async_copy(st0, s_hbm.at[pl.ds(base + c0 * _CH, _CH)], stsem0)

        @pl.when(t < _NCHUNK // 2 - 1)
        def _():
            build_idx(c0 + 2, gidx0, ab0)
            pltpu.async_copy(emb_hbm.at[gidx0], rows0, sem0)
        pltpu.make_async_copy(emb_hbm.at[gidx1], rows1, sem1).wait()

        @pl.when(t > 0)
        def _():
            pltpu.make_async_copy(st1, s_hbm.at[pl.ds(0, _CH)], stsem1).wait()
        compute(rows1, ab1, st1)
        pltpu.async_copy(st1, s_hbm.at[pl.ds(base + (c0 + 1) * _CH, _CH)],
                         stsem1)
        return 0
    lax.fori_loop(0, _NCHUNK // 2, outer, 0)

    pltpu.make_async_copy(st0, s_hbm.at[pl.ds(0, _CH)], stsem0).wait()
    pltpu.make_async_copy(st1, s_hbm.at[pl.ds(0, _CH)], stsem1).wait()


_sc_gather = pl.kernel(
    _sc_body,
    out_type=(jax.ShapeDtypeStruct((_B * _N, _F), jnp.float32),
              jax.ShapeDtypeStruct((_B * _N, 2 * _F), jnp.float32)),
    mesh=_mesh,
    compiler_params=pltpu.CompilerParams(needs_layout_passes=False),
    scratch_types=[
        pltpu.VMEM((_N,), jnp.int32),
        pltpu.VMEM((2, 128), jnp.int32),
        pltpu.VMEM((_RPW * _MC,), jnp.int32),
        pltpu.VMEM((_CH * _MC,), jnp.int32),
        pltpu.VMEM((_CH * _MC,), jnp.int32),
        pltpu.VMEM((_CH * _MC, _F), jnp.float32),
        pltpu.VMEM((_CH * _MC, _F), jnp.float32),
        pltpu.VMEM((2 * _L,), jnp.float32),
        pltpu.VMEM((2 * _L,), jnp.float32),
        pltpu.VMEM((_CH, 2 * _F), jnp.float32),
        pltpu.VMEM((_CH, 2 * _F), jnp.float32),
        pltpu.SemaphoreType.DMA,
        pltpu.SemaphoreType.DMA,
        pltpu.SemaphoreType.DMA,
        pltpu.SemaphoreType.DMA,
    ],
)


def _tc_body(p_ref, s_ref, w0_ref, w1_ref, w2_ref, bc_ref,
             whl_ref, bhl_ref, out_ref, pooled_ref):
    bidx = pl.program_id(0)
    acc = jnp.dot(p_ref[...], w0_ref[...], preferred_element_type=jnp.float32)
    acc += jnp.dot(s_ref[:, :_F], w1_ref[...],
                   preferred_element_type=jnp.float32)
    acc += jnp.dot(s_ref[:, _F:], w2_ref[...],
                   preferred_element_type=jnp.float32)
    t = jnp.tanh(acc + bc_ref[...])
    pooled_ref[pl.ds(bidx, 1), :] = jnp.max(t, axis=0, keepdims=True)

    @pl.when(bidx == _B - 1)
    def _():
        out_ref[...] = lax.dot_general(
            pooled_ref[...], whl_ref[...],
            (((1,), (1,)), ((), ())),
            preferred_element_type=jnp.float32) + bhl_ref[...]


def _tc_conv(p, s, w0, w1, w2, bc, whl, bhl):
    return pl.pallas_call(
        _tc_body,
        grid=(_B,),
        in_specs=[
            pl.BlockSpec((_N, _F), lambda b: (b, 0)),
            pl.BlockSpec((_N, 2 * _F), lambda b: (b, 0)),
            pl.BlockSpec((_F, _CONV), lambda b: (0, 0)),
            pl.BlockSpec((_F, _CONV), lambda b: (0, 0)),
            pl.BlockSpec((_F, _CONV), lambda b: (0, 0)),
            pl.BlockSpec((1, _CONV), lambda b: (0, 0)),
            pl.BlockSpec((_NL, _CONV), lambda b: (0, 0)),
            pl.BlockSpec((1, _NL), lambda b: (0, 0)),
        ],
        out_specs=pl.BlockSpec((_B, _NL), lambda b: (0, 0)),
        out_shape=jax.ShapeDtypeStruct((_B, _NL), jnp.float32),
        scratch_shapes=[pltpu.VMEM((_B, _CONV), jnp.float32)],
    )(p, s, w0, w1, w2, bc, whl, bhl)


def kernel(node, children, emb, w_t, w_l, w_r, b_conv, w_hl, b_hl):
    node = node.astype(jnp.int32)
    children = children.astype(jnp.int32)
    ch_flat = children.reshape(_B, _N * _MC)
    pnode = node.reshape(_NW, 2, 128)
    p, s = _sc_gather(node, pnode, ch_flat, emb)
    w_flat = jnp.concatenate([w_t, w_r, w_l], axis=0)
    w0, w1, w2 = w_flat[0::3], w_flat[1::3], w_flat[2::3]
    return _tc_conv(p, s, w0, w1, w2,
                    b_conv.reshape(1, _CONV), w_hl, b_hl.reshape(1, _NL))
